# Initial kernel scaffold; baseline (speedup 1.0000x reference)
import jax
import jax.numpy as jnp
from jax.experimental import pallas as pl


def kernel(node_features, edges, edge_weights, params):
    raise NotImplementedError("write your pallas kernel here")


# speedup vs baseline: 1.3612x; 1.3612x over previous
import functools

import jax
import jax.numpy as jnp
from jax import lax
from jax.experimental import pallas as pl
from jax.experimental.pallas import tpu as pltpu
from jax.experimental.pallas import tpu_sc as plsc

P_LEN = 50
PCH = 26
PCH_PAD = 32
U1 = 64
U2 = 128
HID = 128


def _bf(x):
    return x.astype(jnp.bfloat16)


def _split(x):
    hi = x.astype(jnp.bfloat16)
    lo = (x - hi.astype(jnp.float32)).astype(jnp.bfloat16)
    return hi, lo


def _dot3(a, bh, bl):
    ah, al = _split(a)
    return (jnp.dot(ah, bh, preferred_element_type=jnp.float32)
            + jnp.dot(ah, bl, preferred_element_type=jnp.float32)
            + jnp.dot(al, bh, preferred_element_type=jnp.float32))


def _gelu(x):
    return 0.5 * x * (1.0 + lax.erf(x * (2.0 ** -0.5)))


def _sig(x):
    return jax.nn.sigmoid(x)


def _fold_ffn(p):
    s = p["bn_g"] / jnp.sqrt(p["bn_v"] + 1e-3)
    t = p["bn_b"] - p["bn_m"] * s
    W = p["W"] * s[:, None]
    b = p["b"] + t @ p["W"]
    return W, b[None, :]


def _lstm_kernel(x_ref, w1h, w1l, u1h, u1l, b1, w2h, w2l, u2h, u2l, b2,
                 wph, wpl, bp, x1_out, y1_out):
    B = x_ref.shape[0]
    h1 = jnp.zeros((B, U1), jnp.float32)
    c1 = jnp.zeros((B, U1), jnp.float32)
    h2 = jnp.zeros((B, U2), jnp.float32)
    c2 = jnp.zeros((B, U2), jnp.float32)
    b1v, b2v = b1[...], b2[...]
    w1hv, w1lv, u1hv, u1lv = w1h[...], w1l[...], u1h[...], u1l[...]
    w2hv, w2lv, u2hv, u2lv = w2h[...], w2l[...], u2h[...], u2l[...]
    for t in range(P_LEN):
        xt = x_ref[:, t * PCH_PAD:(t + 1) * PCH_PAD]
        z1 = _dot3(xt, w1hv, w1lv) + _dot3(h1, u1hv, u1lv) + b1v
        c1 = _sig(z1[:, U1:2 * U1]) * c1 + _sig(z1[:, :U1]) * jnp.tanh(z1[:, 2 * U1:3 * U1])
        h1 = _sig(z1[:, 3 * U1:]) * jnp.tanh(c1)
        z2 = _dot3(h1, w2hv, w2lv) + _dot3(h2, u2hv, u2lv) + b2v
        c2 = _sig(z2[:, U2:2 * U2]) * c2 + _sig(z2[:, :U2]) * jnp.tanh(z2[:, 2 * U2:3 * U2])
        h2 = _sig(z2[:, 3 * U2:]) * jnp.tanh(c2)
    x1_out[...] = h2
    y1_out[...] = _gelu(_dot3(h2, wph[...], wpl[...]) + bp[...])


def _run_lstm(x2d, w1, u1, b1, w2, u2, b2, wp, bp):
    N = x2d.shape[0]
    B = 1000 if N % 1000 == 0 else N
    nb = N // B
    D = P_LEN * PCH_PAD

    def wspec(shape):
        return pl.BlockSpec(shape, lambda i: (0, 0))

    return pl.pallas_call(
        _lstm_kernel,
        grid=(nb,),
        in_specs=[
            pl.BlockSpec((B, D), lambda i: (i, 0)),
            wspec((PCH_PAD, 4 * U1)), wspec((PCH_PAD, 4 * U1)),
            wspec((U1, 4 * U1)), wspec((U1, 4 * U1)), wspec((1, 4 * U1)),
            wspec((U1, 4 * U2)), wspec((U1, 4 * U2)),
            wspec((U2, 4 * U2)), wspec((U2, 4 * U2)), wspec((1, 4 * U2)),
            wspec((HID, HID)), wspec((HID, HID)), wspec((1, HID)),
        ],
        out_specs=[pl.BlockSpec((B, HID), lambda i: (i, 0))] * 2,
        out_shape=[jax.ShapeDtypeStruct((N, HID), jnp.float32)] * 2,
    )(x2d, *w1, *u1, b1, *w2, *u2, b2, *wp, bp)


def _spmm(y, src, dst, ewn):
    N = y.shape[0]
    msgs = jnp.take(y, src, axis=0) * ewn[:, None]
    agg = jax.ops.segment_sum(msgs, dst, num_segments=N)
    return agg, jnp.zeros_like(agg)


def _update_kernel(x_ref, a0_ref, a1_ref, wuxh, wuxl, wuah, wual, bu,
                   wph, wpl, bp, x2_out, y2_out):
    agg = a0_ref[...] + a1_ref[...]
    z = (_dot3(x_ref[...], wuxh[...], wuxl[...])
         + _dot3(agg, wuah[...], wual[...]) + bu[...])
    u = _gelu(z)
    s = jnp.maximum(jnp.sum(u * u, axis=1, keepdims=True), 1e-12)
    x2 = u * lax.rsqrt(s)
    x2_out[...] = x2
    y2_out[...] = _gelu(_dot3(x2, wph[...], wpl[...]) + bp[...])


def _run_update(x, a0, a1, wux, wua, bu, wp, bp):
    N = x.shape[0]
    B = 1000 if N % 1000 == 0 else N
    nb = N // B

    def wspec(shape):
        return pl.BlockSpec(shape, lambda i: (0, 0))

    bspec = pl.BlockSpec((B, HID), lambda i: (i, 0))
    return pl.pallas_call(
        _update_kernel,
        grid=(nb,),
        in_specs=[bspec, bspec, bspec,
                  wspec((HID, HID)), wspec((HID, HID)),
                  wspec((HID, HID)), wspec((HID, HID)), wspec((1, HID)),
                  wspec((HID, HID)), wspec((HID, HID)), wspec((1, HID))],
        out_specs=[bspec] * 2,
        out_shape=[jax.ShapeDtypeStruct((N, HID), jnp.float32)] * 2,
    )(x, a0, a1, *wux, *wua, bu, *wp, bp)


def _final_kernel(x_ref, a0_ref, a1_ref, wuxh, wuxl, wuah, wual, bu,
                  wpoh, wpol, bpo, wlh, wll, bl, out_ref):
    agg = a0_ref[...] + a1_ref[...]
    z = (_dot3(x_ref[...], wuxh[...], wuxl[...])
         + _dot3(agg, wuah[...], wual[...]) + bu[...])
    u = _gelu(z)
    s = jnp.maximum(jnp.sum(u * u, axis=1, keepdims=True), 1e-12)
    x3 = u * lax.rsqrt(s)
    emb = _gelu(_dot3(x3, wpoh[...], wpol[...]) + bpo[...])
    out_ref[...] = _dot3(emb, wlh[...], wll[...]) + bl[...]


def _run_final(x, a0, a1, wux, wua, bu, wpo, bpo, wl, bl, ncls):
    N = x.shape[0]
    B = 1000 if N % 1000 == 0 else N
    nb = N // B

    def wspec(shape):
        return pl.BlockSpec(shape, lambda i: (0, 0))

    bspec = pl.BlockSpec((B, HID), lambda i: (i, 0))
    return pl.pallas_call(
        _final_kernel,
        grid=(nb,),
        in_specs=[bspec, bspec, bspec,
                  wspec((HID, HID)), wspec((HID, HID)),
                  wspec((HID, HID)), wspec((HID, HID)), wspec((1, HID)),
                  wspec((HID, HID)), wspec((HID, HID)), wspec((1, HID)),
                  wspec((HID, ncls)), wspec((HID, ncls)), wspec((1, ncls))],
        out_specs=pl.BlockSpec((B, ncls), lambda i: (i, 0)),
        out_shape=jax.ShapeDtypeStruct((N, ncls), jnp.float32),
    )(x, a0, a1, *wux, *wua, bu, *wpo, bpo, *wl, bl)


def kernel(node_features, edges, edge_weights, params):
    edges = edges.astype(jnp.int32)
    N = node_features.shape[0]
    ewn = edge_weights / jnp.sum(edge_weights)

    x = node_features.astype(jnp.float32)
    x = jnp.pad(x, ((0, 0), (0, 0), (0, PCH_PAD - PCH)))
    x2d = x.reshape(N, P_LEN * PCH_PAD)

    p = params
    w1 = _split(jnp.pad(p["lstm1"]["W"], ((0, PCH_PAD - PCH), (0, 0))))
    u1 = _split(p["lstm1"]["U"])
    b1 = p["lstm1"]["b"][None, :]
    w2 = _split(p["lstm2"]["W"])
    u2 = _split(p["lstm2"]["U"])
    b2 = p["lstm2"]["b"][None, :]
    wp1, bp1 = _fold_ffn(p["conv1"]["prepare"])
    wu1, bu1 = _fold_ffn(p["conv1"]["update"])
    wp2, bp2 = _fold_ffn(p["conv2"]["prepare"])
    wu2, bu2 = _fold_ffn(p["conv2"]["update"])
    wpo, bpo = _fold_ffn(p["post"])
    wl = _split(p["logits_W"])
    bl = p["logits_b"][None, :]
    ncls = p["logits_W"].shape[1]

    x1, y1 = _run_lstm(x2d, w1, u1, b1, w2, u2, b2,
                       _split(wp1), bp1)
    a0, a1 = _spmm(y1, edges[1], edges[0], ewn)
    x2, y2 = _run_update(x1, a0, a1, _split(wu1[:HID]), _split(wu1[HID:]), bu1,
                         _split(wp2), bp2)
    a0, a1 = _spmm(y2, edges[1], edges[0], ewn)
    return _run_final(x2, a0, a1, _split(wu2[:HID]), _split(wu2[HID:]), bu2,
                      _split(wpo), bpo, wl, bl, ncls)


# speedup vs baseline: 2.1821x; 1.6031x over previous
import functools

import jax
import jax.numpy as jnp
from jax import lax
from jax.experimental import pallas as pl
from jax.experimental.pallas import tpu as pltpu
from jax.experimental.pallas import tpu_sc as plsc

P_LEN = 50
PCH = 26
PCH_PAD = 32
U1 = 64
U2 = 128
HID = 128


def _split(x):
    hi = x.astype(jnp.bfloat16)
    lo = (x - hi.astype(jnp.float32)).astype(jnp.bfloat16)
    return hi, lo


def _dot3(a, bh, bl):
    ah, al = _split(a)
    return (jnp.dot(ah, bh, preferred_element_type=jnp.float32)
            + jnp.dot(ah, bl, preferred_element_type=jnp.float32)
            + jnp.dot(al, bh, preferred_element_type=jnp.float32))


def _gelu(x):
    return 0.5 * x * (1.0 + lax.erf(x * (2.0 ** -0.5)))


def _sig(x):
    return jax.nn.sigmoid(x)


def _fold_ffn(p):
    s = p["bn_g"] / jnp.sqrt(p["bn_v"] + 1e-3)
    t = p["bn_b"] - p["bn_m"] * s
    W = p["W"] * s[:, None]
    b = p["b"] + t @ p["W"]
    return W, b[None, :]


def _lstm_kernel(x_ref, w1h, w1l, u1h, u1l, b1, w2h, w2l, u2h, u2l, b2,
                 wph, wpl, bp, x1_out, y1_out):
    B = x_ref.shape[0]
    h1 = jnp.zeros((B, U1), jnp.float32)
    c1 = jnp.zeros((B, U1), jnp.float32)
    h2 = jnp.zeros((B, U2), jnp.float32)
    c2 = jnp.zeros((B, U2), jnp.float32)
    b1v, b2v = b1[...], b2[...]
    w1hv, w1lv, u1hv, u1lv = w1h[...], w1l[...], u1h[...], u1l[...]
    w2hv, w2lv, u2hv, u2lv = w2h[...], w2l[...], u2h[...], u2l[...]
    for t in range(P_LEN):
        xt = x_ref[:, t * PCH_PAD:(t + 1) * PCH_PAD]
        z1 = _dot3(xt, w1hv, w1lv) + _dot3(h1, u1hv, u1lv) + b1v
        c1 = _sig(z1[:, U1:2 * U1]) * c1 + _sig(z1[:, :U1]) * jnp.tanh(z1[:, 2 * U1:3 * U1])
        h1 = _sig(z1[:, 3 * U1:]) * jnp.tanh(c1)
        z2 = _dot3(h1, w2hv, w2lv) + _dot3(h2, u2hv, u2lv) + b2v
        c2 = _sig(z2[:, U2:2 * U2]) * c2 + _sig(z2[:, :U2]) * jnp.tanh(z2[:, 2 * U2:3 * U2])
        h2 = _sig(z2[:, 3 * U2:]) * jnp.tanh(c2)
    x1_out[...] = h2
    y1_out[...] = _gelu(_dot3(h2, wph[...], wpl[...]) + bp[...])


def _run_lstm(x2d, w1, u1, b1, w2, u2, b2, wp, bp):
    N = x2d.shape[0]
    B = 1000 if N % 1000 == 0 else N
    nb = N // B
    D = P_LEN * PCH_PAD

    def wspec(shape):
        return pl.BlockSpec(shape, lambda i: (0, 0))

    return pl.pallas_call(
        _lstm_kernel,
        grid=(nb,),
        in_specs=[
            pl.BlockSpec((B, D), lambda i: (i, 0)),
            wspec((PCH_PAD, 4 * U1)), wspec((PCH_PAD, 4 * U1)),
            wspec((U1, 4 * U1)), wspec((U1, 4 * U1)), wspec((1, 4 * U1)),
            wspec((U1, 4 * U2)), wspec((U1, 4 * U2)),
            wspec((U2, 4 * U2)), wspec((U2, 4 * U2)), wspec((1, 4 * U2)),
            wspec((HID, HID)), wspec((HID, HID)), wspec((1, HID)),
        ],
        out_specs=[pl.BlockSpec((B, HID), lambda i: (i, 0))] * 2,
        out_shape=[jax.ShapeDtypeStruct((N, HID), jnp.float32)] * 2,
    )(x2d, *w1, *u1, b1, *w2, *u2, b2, *wp, bp)


_SC_NC = 2
_SC_NS = 16
_SC_NW = _SC_NC * _SC_NS
_SC_K = 128
_SC_SK = 8


@functools.lru_cache(maxsize=None)
def _sc_spmm_make(N, E):
    epw = E // _SC_NW
    nch = epw // _SC_K
    nsk = nch // _SC_SK
    zr = 128
    npad = ((N + zr * _SC_NS - 1) // (zr * _SC_NS)) * (zr * _SC_NS)
    rps = npad // _SC_NS
    nz = rps // zr
    mesh = plsc.VectorSubcoreMesh(core_axis_name="c", subcore_axis_name="s")

    @functools.partial(
        pl.kernel,
        out_type=jax.ShapeDtypeStruct((_SC_NC, npad, HID), jnp.float32),
        mesh=mesh,
        scratch_types=[
            pltpu.VMEM((_SC_SK, _SC_K), jnp.int32),
            pltpu.VMEM((_SC_SK, _SC_K), jnp.int32),
            pltpu.VMEM((_SC_SK, _SC_K), jnp.float32),
            pltpu.VMEM((_SC_K, HID), jnp.float32),
            pltpu.VMEM((zr, HID), jnp.float32),
            pltpu.VMEM_SHARED((npad, HID), jnp.float32),
            pltpu.SemaphoreType.DMA,
        ],
    )
    def spmm(y_hbm, src_hbm, dst_hbm, ew_hbm, out_hbm,
             src_v, dst_v, ew_v, rows_v, zbuf, acc, sem):
        cid = lax.axis_index("c")
        sid = lax.axis_index("s")
        wid = sid * _SC_NC + cid

        zv = jnp.zeros((16,), jnp.float32)

        def zrow(r, carry):
            for k8 in range(HID // 16):
                zbuf[r, pl.ds(16 * k8, 16)] = zv
            return carry

        lax.fori_loop(0, zr, zrow, 0)
        for i in range(nz):
            pltpu.sync_copy(zbuf, acc.at[pl.ds(sid * rps + i * zr, zr)])
        plsc.subcore_barrier()

        def superchunk(jj, carry):
            pltpu.sync_copy(src_hbm.at[wid, pl.ds(jj * _SC_SK, _SC_SK)], src_v)
            pltpu.sync_copy(dst_hbm.at[wid, pl.ds(jj * _SC_SK, _SC_SK)], dst_v)
            pltpu.sync_copy(ew_hbm.at[wid, pl.ds(jj * _SC_SK, _SC_SK)], ew_v)

            def chunk(j, c1):
                pltpu.async_copy(y_hbm.at[src_v.at[j]], rows_v, sem).wait()

                def edge16(g, c2):
                    wg = ew_v[j, pl.ds(16 * g, 16)]
                    for m in range(16):
                        w = lax.gather(
                            wg, jnp.full((16, 1), m, jnp.int32),
                            lax.GatherDimensionNumbers(
                                offset_dims=(), collapsed_slice_dims=(0,),
                                start_index_map=(0,)),
                            (1,),
                            mode=lax.GatherScatterMode.PROMISE_IN_BOUNDS)
                        e = g * 16 + m
                        for k8 in range(HID // 16):
                            s = pl.ds(16 * k8, 16)
                            rows_v[e, s] = rows_v[e, s] * w
                    return c2

                lax.fori_loop(0, _SC_K // 16, edge16, 0)
                pltpu.sync_copy(rows_v, acc.at[dst_v.at[j]], add=True)
                return c1

            lax.fori_loop(0, _SC_SK, chunk, 0)
            return carry

        lax.fori_loop(0, nsk, superchunk, 0)
        plsc.subcore_barrier()

        for i in range(nz):
            r0 = sid * rps + i * zr
            pltpu.sync_copy(acc.at[pl.ds(r0, zr)],
                            out_hbm.at[cid, pl.ds(r0, zr)])

    return spmm


def _spmm(y, src_r, dst_r, ew_r):
    N = y.shape[0]
    parts = _sc_spmm_make(N, src_r.size)(y, src_r, dst_r, ew_r)
    return parts[0, :N], parts[1, :N]


def _update_kernel(x_ref, a0_ref, a1_ref, wuxh, wuxl, wuah, wual, bu,
                   wph, wpl, bp, x2_out, y2_out):
    agg = a0_ref[...] + a1_ref[...]
    z = (_dot3(x_ref[...], wuxh[...], wuxl[...])
         + _dot3(agg, wuah[...], wual[...]) + bu[...])
    u = _gelu(z)
    s = jnp.maximum(jnp.sum(u * u, axis=1, keepdims=True), 1e-12)
    x2 = u * lax.rsqrt(s)
    x2_out[...] = x2
    y2_out[...] = _gelu(_dot3(x2, wph[...], wpl[...]) + bp[...])


def _run_update(x, a0, a1, wux, wua, bu, wp, bp):
    N = x.shape[0]
    B = 1000 if N % 1000 == 0 else N
    nb = N // B

    def wspec(shape):
        return pl.BlockSpec(shape, lambda i: (0, 0))

    bspec = pl.BlockSpec((B, HID), lambda i: (i, 0))
    return pl.pallas_call(
        _update_kernel,
        grid=(nb,),
        in_specs=[bspec, bspec, bspec,
                  wspec((HID, HID)), wspec((HID, HID)),
                  wspec((HID, HID)), wspec((HID, HID)), wspec((1, HID)),
                  wspec((HID, HID)), wspec((HID, HID)), wspec((1, HID))],
        out_specs=[bspec] * 2,
        out_shape=[jax.ShapeDtypeStruct((N, HID), jnp.float32)] * 2,
    )(x, a0, a1, *wux, *wua, bu, *wp, bp)


def _final_kernel(x_ref, a0_ref, a1_ref, wuxh, wuxl, wuah, wual, bu,
                  wpoh, wpol, bpo, wlh, wll, bl, out_ref):
    agg = a0_ref[...] + a1_ref[...]
    z = (_dot3(x_ref[...], wuxh[...], wuxl[...])
         + _dot3(agg, wuah[...], wual[...]) + bu[...])
    u = _gelu(z)
    s = jnp.maximum(jnp.sum(u * u, axis=1, keepdims=True), 1e-12)
    x3 = u * lax.rsqrt(s)
    emb = _gelu(_dot3(x3, wpoh[...], wpol[...]) + bpo[...])
    out_ref[...] = _dot3(emb, wlh[...], wll[...]) + bl[...]


def _run_final(x, a0, a1, wux, wua, bu, wpo, bpo, wl, bl, ncls):
    N = x.shape[0]
    B = 1000 if N % 1000 == 0 else N
    nb = N // B

    def wspec(shape):
        return pl.BlockSpec(shape, lambda i: (0, 0))

    bspec = pl.BlockSpec((B, HID), lambda i: (i, 0))
    return pl.pallas_call(
        _final_kernel,
        grid=(nb,),
        in_specs=[bspec, bspec, bspec,
                  wspec((HID, HID)), wspec((HID, HID)),
                  wspec((HID, HID)), wspec((HID, HID)), wspec((1, HID)),
                  wspec((HID, HID)), wspec((HID, HID)), wspec((1, HID)),
                  wspec((HID, ncls)), wspec((HID, ncls)), wspec((1, ncls))],
        out_specs=pl.BlockSpec((B, ncls), lambda i: (i, 0)),
        out_shape=jax.ShapeDtypeStruct((N, ncls), jnp.float32),
    )(x, a0, a1, *wux, *wua, bu, *wpo, bpo, *wl, bl)


def kernel(node_features, edges, edge_weights, params):
    edges = edges.astype(jnp.int32)
    N = node_features.shape[0]
    ewn = edge_weights / jnp.sum(edge_weights)
    E = edges.shape[1]
    echunk = _SC_NW * _SC_K * _SC_SK
    epad = (-E) % echunk
    ewn = jnp.pad(ewn, (0, epad))
    src_p = jnp.pad(edges[1], (0, epad))
    dst_p = jnp.pad(edges[0], (0, epad))
    ew_r = ewn.reshape(_SC_NW, -1, _SC_K)
    src_r = src_p.reshape(_SC_NW, -1, _SC_K)
    dst_r = dst_p.reshape(_SC_NW, -1, _SC_K)

    x = node_features.astype(jnp.float32)
    x = jnp.pad(x, ((0, 0), (0, 0), (0, PCH_PAD - PCH)))
    x2d = x.reshape(N, P_LEN * PCH_PAD)

    p = params
    w1 = _split(jnp.pad(p["lstm1"]["W"], ((0, PCH_PAD - PCH), (0, 0))))
    u1 = _split(p["lstm1"]["U"])
    b1 = p["lstm1"]["b"][None, :]
    w2 = _split(p["lstm2"]["W"])
    u2 = _split(p["lstm2"]["U"])
    b2 = p["lstm2"]["b"][None, :]
    wp1, bp1 = _fold_ffn(p["conv1"]["prepare"])
    wu1, bu1 = _fold_ffn(p["conv1"]["update"])
    wp2, bp2 = _fold_ffn(p["conv2"]["prepare"])
    wu2, bu2 = _fold_ffn(p["conv2"]["update"])
    wpo, bpo = _fold_ffn(p["post"])
    wl = _split(p["logits_W"])
    bl = p["logits_b"][None, :]
    ncls = p["logits_W"].shape[1]

    x1, y1 = _run_lstm(x2d, w1, u1, b1, w2, u2, b2, _split(wp1), bp1)
    a0, a1 = _spmm(y1, src_r, dst_r, ew_r)
    x2, y2 = _run_update(x1, a0, a1, _split(wu1[:HID]), _split(wu1[HID:]), bu1,
                         _split(wp2), bp2)
    a0, a1 = _spmm(y2, src_r, dst_r, ew_r)
    return _run_final(x2, a0, a1, _split(wu2[:HID]), _split(wu2[HID:]), bu2,
                      _split(wpo), bpo, wl, bl, ncls)


# speedup vs baseline: 2.7523x; 1.2613x over previous
import functools

import jax
import jax.numpy as jnp
from jax import lax
from jax.experimental import pallas as pl
from jax.experimental.pallas import tpu as pltpu
from jax.experimental.pallas import tpu_sc as plsc

P_LEN = 50
PCH = 26
PCH_PAD = 32
U1 = 64
U2 = 128
HID = 128


def _split(x):
    hi = x.astype(jnp.bfloat16)
    lo = (x - hi.astype(jnp.float32)).astype(jnp.bfloat16)
    return hi, lo


def _dot3(a, bh, bl):
    ah, al = _split(a)
    return (jnp.dot(ah, bh, preferred_element_type=jnp.float32)
            + jnp.dot(ah, bl, preferred_element_type=jnp.float32)
            + jnp.dot(al, bh, preferred_element_type=jnp.float32))


def _gelu(x):
    return 0.5 * x * (1.0 + lax.erf(x * (2.0 ** -0.5)))


def _sig(x):
    return jax.nn.sigmoid(x)


def _fold_ffn(p):
    s = p["bn_g"] / jnp.sqrt(p["bn_v"] + 1e-3)
    t = p["bn_b"] - p["bn_m"] * s
    W = p["W"] * s[:, None]
    b = p["b"] + t @ p["W"]
    return W, b[None, :]


def _dotp(a, b):
    return jnp.dot(a, b, preferred_element_type=jnp.float32)


def _lstm_kernel(x_ref, wz1h, wz1l, b1, wz2h, wz2l, b2,
                 wph, wpl, bp, x1_out, y1_out):
    B = x_ref.shape[0]
    h1 = jnp.zeros((B, U1), jnp.float32)
    c1 = jnp.zeros((B, U1), jnp.float32)
    h2 = jnp.zeros((B, U2), jnp.float32)
    c2 = jnp.zeros((B, U2), jnp.float32)
    b1v, b2v = b1[...], b2[...]
    wz1hv, wz1lv = wz1h[...], wz1l[...]
    wz2hv, wz2lv = wz2h[...], wz2l[...]
    xz = jnp.zeros((B, PCH), jnp.bfloat16)
    for t in range(P_LEN):
        xt = x_ref[:, t * PCH:(t + 1) * PCH]
        h1h, h1l = _split(h1)
        ah1 = jnp.concatenate([xt, h1h], axis=1)
        al1 = jnp.concatenate([xz, h1l], axis=1)
        z1 = _dotp(ah1, wz1hv) + _dotp(ah1, wz1lv) + _dotp(al1, wz1hv) + b1v
        c1 = _sig(z1[:, U1:2 * U1]) * c1 + _sig(z1[:, :U1]) * jnp.tanh(z1[:, 2 * U1:3 * U1])
        h1 = _sig(z1[:, 3 * U1:]) * jnp.tanh(c1)
        h1h, h1l = _split(h1)
        h2h, h2l = _split(h2)
        ah2 = jnp.concatenate([h1h, h2h], axis=1)
        al2 = jnp.concatenate([h1l, h2l], axis=1)
        z2 = _dotp(ah2, wz2hv) + _dotp(ah2, wz2lv) + _dotp(al2, wz2hv) + b2v
        c2 = _sig(z2[:, U2:2 * U2]) * c2 + _sig(z2[:, :U2]) * jnp.tanh(z2[:, 2 * U2:3 * U2])
        h2 = _sig(z2[:, 3 * U2:]) * jnp.tanh(c2)
    x1_out[...] = h2
    y1_out[...] = _gelu(_dot3(h2, wph[...], wpl[...]) + bp[...])


def _run_lstm(x2d, wz1, b1, wz2, b2, wp, bp):
    N = x2d.shape[0]
    B = 1000 if N % 1000 == 0 else N
    nb = N // B
    D = P_LEN * PCH

    def wspec(shape):
        return pl.BlockSpec(shape, lambda i: (0, 0))

    return pl.pallas_call(
        _lstm_kernel,
        grid=(nb,),
        in_specs=[
            pl.BlockSpec((B, D), lambda i: (i, 0)),
            wspec((PCH + U1, 4 * U1)), wspec((PCH + U1, 4 * U1)), wspec((1, 4 * U1)),
            wspec((U1 + U2, 4 * U2)), wspec((U1 + U2, 4 * U2)), wspec((1, 4 * U2)),
            wspec((HID, HID)), wspec((HID, HID)), wspec((1, HID)),
        ],
        out_specs=[pl.BlockSpec((B, HID), lambda i: (i, 0))] * 2,
        out_shape=[jax.ShapeDtypeStruct((N, HID), jnp.float32)] * 2,
    )(x2d, *wz1, b1, *wz2, b2, *wp, bp)


_SC_NC = 2
_SC_NS = 16
_SC_NW = _SC_NC * _SC_NS
_SC_K = 128
_SC_SK = 8


@functools.lru_cache(maxsize=None)
def _sc_spmm_make(N, E):
    epw = E // _SC_NW
    nch = epw // _SC_K
    nsk = nch // _SC_SK
    zr = 128
    npad = ((N + zr * _SC_NS - 1) // (zr * _SC_NS)) * (zr * _SC_NS)
    rps = npad // _SC_NS
    nz = rps // zr
    mesh = plsc.VectorSubcoreMesh(core_axis_name="c", subcore_axis_name="s")

    @functools.partial(
        pl.kernel,
        out_type=jax.ShapeDtypeStruct((_SC_NC, npad, HID), jnp.float32),
        mesh=mesh,
        scratch_types=[
            pltpu.VMEM((_SC_SK, _SC_K), jnp.int32),
            pltpu.VMEM((_SC_SK, _SC_K), jnp.int32),
            pltpu.VMEM((_SC_SK, _SC_K), jnp.float32),
            pltpu.VMEM((_SC_K, HID), jnp.float32),
            pltpu.VMEM((zr, HID), jnp.float32),
            pltpu.VMEM_SHARED((npad, HID), jnp.float32),
            pltpu.SemaphoreType.DMA,
        ],
    )
    def spmm(y_hbm, src_hbm, dst_hbm, ew_hbm, out_hbm,
             src_v, dst_v, ew_v, rows_v, zbuf, acc, sem):
        cid = lax.axis_index("c")
        sid = lax.axis_index("s")
        wid = sid * _SC_NC + cid

        zv = jnp.zeros((16,), jnp.float32)

        def zrow(r, carry):
            for k8 in range(HID // 16):
                zbuf[r, pl.ds(16 * k8, 16)] = zv
            return carry

        lax.fori_loop(0, zr, zrow, 0)
        for i in range(nz):
            pltpu.sync_copy(zbuf, acc.at[pl.ds(sid * rps + i * zr, zr)])
        plsc.subcore_barrier()

        def superchunk(jj, carry):
            pltpu.sync_copy(src_hbm.at[wid, pl.ds(jj * _SC_SK, _SC_SK)], src_v)
            pltpu.sync_copy(dst_hbm.at[wid, pl.ds(jj * _SC_SK, _SC_SK)], dst_v)
            pltpu.sync_copy(ew_hbm.at[wid, pl.ds(jj * _SC_SK, _SC_SK)], ew_v)

            def chunk(j, c1):
                pltpu.async_copy(y_hbm.at[src_v.at[j]], rows_v, sem).wait()

                def edge16(g, c2):
                    wg = ew_v[j, pl.ds(16 * g, 16)]
                    for m in range(16):
                        w = lax.gather(
                            wg, jnp.full((16, 1), m, jnp.int32),
                            lax.GatherDimensionNumbers(
                                offset_dims=(), collapsed_slice_dims=(0,),
                                start_index_map=(0,)),
                            (1,),
                            mode=lax.GatherScatterMode.PROMISE_IN_BOUNDS)
                        e = g * 16 + m
                        for k8 in range(HID // 16):
                            s = pl.ds(16 * k8, 16)
                            rows_v[e, s] = rows_v[e, s] * w
                    return c2

                lax.fori_loop(0, _SC_K // 16, edge16, 0)
                pltpu.sync_copy(rows_v, acc.at[dst_v.at[j]], add=True)
                return c1

            lax.fori_loop(0, _SC_SK, chunk, 0)
            return carry

        lax.fori_loop(0, nsk, superchunk, 0)
        plsc.subcore_barrier()

        for i in range(nz):
            r0 = sid * rps + i * zr
            pltpu.sync_copy(acc.at[pl.ds(r0, zr)],
                            out_hbm.at[cid, pl.ds(r0, zr)])

    return spmm


def _spmm(y, src_r, dst_r, ew_r):
    N = y.shape[0]
    parts = _sc_spmm_make(N, src_r.size)(y, src_r, dst_r, ew_r)
    return parts[0, :N], parts[1, :N]


def _update_kernel(x_ref, a0_ref, a1_ref, wuxh, wuxl, wuah, wual, bu,
                   wph, wpl, bp, x2_out, y2_out):
    agg = a0_ref[...] + a1_ref[...]
    z = (_dot3(x_ref[...], wuxh[...], wuxl[...])
         + _dot3(agg, wuah[...], wual[...]) + bu[...])
    u = _gelu(z)
    s = jnp.maximum(jnp.sum(u * u, axis=1, keepdims=True), 1e-12)
    x2 = u * lax.rsqrt(s)
    x2_out[...] = x2
    y2_out[...] = _gelu(_dot3(x2, wph[...], wpl[...]) + bp[...])


def _run_update(x, a0, a1, wux, wua, bu, wp, bp):
    N = x.shape[0]
    B = 1000 if N % 1000 == 0 else N
    nb = N // B

    def wspec(shape):
        return pl.BlockSpec(shape, lambda i: (0, 0))

    bspec = pl.BlockSpec((B, HID), lambda i: (i, 0))
    return pl.pallas_call(
        _update_kernel,
        grid=(nb,),
        in_specs=[bspec, bspec, bspec,
                  wspec((HID, HID)), wspec((HID, HID)),
                  wspec((HID, HID)), wspec((HID, HID)), wspec((1, HID)),
                  wspec((HID, HID)), wspec((HID, HID)), wspec((1, HID))],
        out_specs=[bspec] * 2,
        out_shape=[jax.ShapeDtypeStruct((N, HID), jnp.float32)] * 2,
    )(x, a0, a1, *wux, *wua, bu, *wp, bp)


def _final_kernel(x_ref, a0_ref, a1_ref, wuxh, wuxl, wuah, wual, bu,
                  wpoh, wpol, bpo, wlh, wll, bl, out_ref):
    agg = a0_ref[...] + a1_ref[...]
    z = (_dot3(x_ref[...], wuxh[...], wuxl[...])
         + _dot3(agg, wuah[...], wual[...]) + bu[...])
    u = _gelu(z)
    s = jnp.maximum(jnp.sum(u * u, axis=1, keepdims=True), 1e-12)
    x3 = u * lax.rsqrt(s)
    emb = _gelu(_dot3(x3, wpoh[...], wpol[...]) + bpo[...])
    out_ref[...] = _dot3(emb, wlh[...], wll[...]) + bl[...]


def _run_final(x, a0, a1, wux, wua, bu, wpo, bpo, wl, bl, ncls):
    N = x.shape[0]
    B = 1000 if N % 1000 == 0 else N
    nb = N // B

    def wspec(shape):
        return pl.BlockSpec(shape, lambda i: (0, 0))

    bspec = pl.BlockSpec((B, HID), lambda i: (i, 0))
    return pl.pallas_call(
        _final_kernel,
        grid=(nb,),
        in_specs=[bspec, bspec, bspec,
                  wspec((HID, HID)), wspec((HID, HID)),
                  wspec((HID, HID)), wspec((HID, HID)), wspec((1, HID)),
                  wspec((HID, HID)), wspec((HID, HID)), wspec((1, HID)),
                  wspec((HID, ncls)), wspec((HID, ncls)), wspec((1, ncls))],
        out_specs=pl.BlockSpec((B, ncls), lambda i: (i, 0)),
        out_shape=jax.ShapeDtypeStruct((N, ncls), jnp.float32),
    )(x, a0, a1, *wux, *wua, bu, *wpo, bpo, *wl, bl)


def kernel(node_features, edges, edge_weights, params):
    edges = edges.astype(jnp.int32)
    N = node_features.shape[0]
    ewn = edge_weights / jnp.sum(edge_weights)
    E = edges.shape[1]
    echunk = _SC_NW * _SC_K * _SC_SK
    epad = (-E) % echunk
    ewn = jnp.pad(ewn, (0, epad))
    src_p = jnp.pad(edges[1], (0, epad))
    dst_p = jnp.pad(edges[0], (0, epad))
    ew_r = ewn.reshape(_SC_NW, -1, _SC_K)
    src_r = src_p.reshape(_SC_NW, -1, _SC_K)
    dst_r = dst_p.reshape(_SC_NW, -1, _SC_K)

    x2d = node_features.reshape(N, P_LEN * PCH).astype(jnp.bfloat16)

    p = params
    wz1 = _split(jnp.concatenate([p["lstm1"]["W"], p["lstm1"]["U"]], axis=0))
    b1 = p["lstm1"]["b"][None, :]
    wz2 = _split(jnp.concatenate([p["lstm2"]["W"], p["lstm2"]["U"]], axis=0))
    b2 = p["lstm2"]["b"][None, :]
    wp1, bp1 = _fold_ffn(p["conv1"]["prepare"])
    wu1, bu1 = _fold_ffn(p["conv1"]["update"])
    wp2, bp2 = _fold_ffn(p["conv2"]["prepare"])
    wu2, bu2 = _fold_ffn(p["conv2"]["update"])
    wpo, bpo = _fold_ffn(p["post"])
    wl = _split(p["logits_W"])
    bl = p["logits_b"][None, :]
    ncls = p["logits_W"].shape[1]

    x1, y1 = _run_lstm(x2d, wz1, b1, wz2, b2, _split(wp1), bp1)
    a0, a1 = _spmm(y1, src_r, dst_r, ew_r)
    x2, y2 = _run_update(x1, a0, a1, _split(wu1[:HID]), _split(wu1[HID:]), bu1,
                         _split(wp2), bp2)
    a0, a1 = _spmm(y2, src_r, dst_r, ew_r)
    return _run_final(x2, a0, a1, _split(wu2[:HID]), _split(wu2[HID:]), bu2,
                      _split(wpo), bpo, wl, bl, ncls)


# speedup vs baseline: 2.9644x; 1.0771x over previous
import functools

import jax
import jax.numpy as jnp
from jax import lax
from jax.experimental import pallas as pl
from jax.experimental.pallas import tpu as pltpu
from jax.experimental.pallas import tpu_sc as plsc

P_LEN = 50
PCH = 26
PCH_PAD = 32
U1 = 64
U2 = 128
HID = 128


def _split(x):
    hi = x.astype(jnp.bfloat16)
    lo = (x - hi.astype(jnp.float32)).astype(jnp.bfloat16)
    return hi, lo


def _dot3(a, bh, bl):
    ah, al = _split(a)
    return (jnp.dot(ah, bh, preferred_element_type=jnp.float32)
            + jnp.dot(ah, bl, preferred_element_type=jnp.float32)
            + jnp.dot(al, bh, preferred_element_type=jnp.float32))


def _gelu(x):
    return 0.5 * x * (1.0 + lax.erf(x * (2.0 ** -0.5)))


def _sig(x):
    return jax.nn.sigmoid(x)


def _fold_ffn(p):
    s = p["bn_g"] / jnp.sqrt(p["bn_v"] + 1e-3)
    t = p["bn_b"] - p["bn_m"] * s
    W = p["W"] * s[:, None]
    b = p["b"] + t @ p["W"]
    return W, b[None, :]


def _dotp(a, b):
    return jnp.dot(a, b, preferred_element_type=jnp.float32)


def _lstm_kernel(x_ref, wz1h, wz1l, b1, wz2h, wz2l, b2,
                 wph, wpl, bp, x1_out, y1_out):
    B = x_ref.shape[0]
    h1 = jnp.zeros((B, U1), jnp.float32)
    c1 = jnp.zeros((B, U1), jnp.float32)
    h2 = jnp.zeros((B, U2), jnp.float32)
    c2 = jnp.zeros((B, U2), jnp.float32)
    b1v, b2v = b1[...], b2[...]
    wz1hv, wz1lv = wz1h[...], wz1l[...]
    wz2hv, wz2lv = wz2h[...], wz2l[...]
    xz = jnp.zeros((B, PCH), jnp.bfloat16)
    for t in range(P_LEN):
        xt = x_ref[:, t * PCH:(t + 1) * PCH]
        h1h, h1l = _split(h1)
        ah1 = jnp.concatenate([xt, h1h], axis=1)
        al1 = jnp.concatenate([xz, h1l], axis=1)
        z1 = _dotp(ah1, wz1hv) + _dotp(ah1, wz1lv) + _dotp(al1, wz1hv) + b1v
        c1 = _sig(z1[:, U1:2 * U1]) * c1 + _sig(z1[:, :U1]) * jnp.tanh(z1[:, 2 * U1:3 * U1])
        h1 = _sig(z1[:, 3 * U1:]) * jnp.tanh(c1)
        h1h, h1l = _split(h1)
        h2h, h2l = _split(h2)
        ah2 = jnp.concatenate([h1h, h2h], axis=1)
        al2 = jnp.concatenate([h1l, h2l], axis=1)
        z2 = _dotp(ah2, wz2hv) + _dotp(ah2, wz2lv) + _dotp(al2, wz2hv) + b2v
        c2 = _sig(z2[:, U2:2 * U2]) * c2 + _sig(z2[:, :U2]) * jnp.tanh(z2[:, 2 * U2:3 * U2])
        h2 = _sig(z2[:, 3 * U2:]) * jnp.tanh(c2)
    x1_out[...] = h2
    y1_out[...] = _gelu(_dot3(h2, wph[...], wpl[...]) + bp[...])


def _run_lstm(x2d, wz1, b1, wz2, b2, wp, bp):
    N = x2d.shape[0]
    B = 1000 if N % 1000 == 0 else N
    nb = N // B
    D = P_LEN * PCH

    def wspec(shape):
        return pl.BlockSpec(shape, lambda i: (0, 0))

    return pl.pallas_call(
        _lstm_kernel,
        grid=(nb,),
        in_specs=[
            pl.BlockSpec((B, D), lambda i: (i, 0)),
            wspec((PCH + U1, 4 * U1)), wspec((PCH + U1, 4 * U1)), wspec((1, 4 * U1)),
            wspec((U1 + U2, 4 * U2)), wspec((U1 + U2, 4 * U2)), wspec((1, 4 * U2)),
            wspec((HID, HID)), wspec((HID, HID)), wspec((1, HID)),
        ],
        out_specs=[pl.BlockSpec((B, HID), lambda i: (i, 0))] * 2,
        out_shape=[jax.ShapeDtypeStruct((N, HID), jnp.float32)] * 2,
    )(x2d, *wz1, b1, *wz2, b2, *wp, bp)


_SC_NC = 2
_SC_NS = 16
_SC_NW = _SC_NC * _SC_NS
_SC_K = 128
_SC_SK = 8


@functools.lru_cache(maxsize=None)
def _sc_spmm_make(N, E):
    epw = E // _SC_NW
    nch = epw // _SC_K
    nsk = nch // _SC_SK
    zr = 64
    npad = ((N + zr * _SC_NS - 1) // (zr * _SC_NS)) * (zr * _SC_NS)
    rps = npad // _SC_NS
    nz = rps // zr
    mesh = plsc.VectorSubcoreMesh(core_axis_name="c", subcore_axis_name="s")

    @functools.partial(
        pl.kernel,
        out_type=jax.ShapeDtypeStruct((_SC_NC, npad, HID), jnp.float32),
        mesh=mesh,
        scratch_types=[
            pltpu.VMEM((_SC_SK, _SC_K), jnp.int32),
            pltpu.VMEM((_SC_SK, _SC_K), jnp.int32),
            pltpu.VMEM((_SC_SK, _SC_K), jnp.float32),
            pltpu.VMEM((2, _SC_K, HID), jnp.float32),
            pltpu.VMEM((zr, HID), jnp.float32),
            pltpu.VMEM_SHARED((npad, HID), jnp.float32),
            pltpu.SemaphoreType.DMA,
            pltpu.SemaphoreType.DMA,
        ],
    )
    def spmm(y_hbm, src_hbm, dst_hbm, ew_hbm, out_hbm,
             src_v, dst_v, ew_v, rows_v, zbuf, acc, gsem, ssem):
        cid = lax.axis_index("c")
        sid = lax.axis_index("s")
        wid = sid * _SC_NC + cid

        zv = jnp.zeros((16,), jnp.float32)

        def zrow(r, carry):
            for k8 in range(HID // 16):
                zbuf[r, pl.ds(16 * k8, 16)] = zv
            return carry

        lax.fori_loop(0, zr, zrow, 0)
        for i in range(nz):
            pltpu.sync_copy(zbuf, acc.at[pl.ds(sid * rps + i * zr, zr)])
        plsc.subcore_barrier()

        def scale(b, j):
            def edge16(g, c2):
                wg = ew_v[j, pl.ds(16 * g, 16)]
                for m in range(16):
                    w = lax.gather(
                        wg, jnp.full((16, 1), m, jnp.int32),
                        lax.GatherDimensionNumbers(
                            offset_dims=(), collapsed_slice_dims=(0,),
                            start_index_map=(0,)),
                        (1,),
                        mode=lax.GatherScatterMode.PROMISE_IN_BOUNDS)
                    e = g * 16 + m
                    for k8 in range(HID // 16):
                        s = pl.ds(16 * k8, 16)
                        rows_v[b, e, s] = rows_v[b, e, s] * w
                return c2

            lax.fori_loop(0, _SC_K // 16, edge16, 0)

        def superchunk(jj, carry):
            pltpu.sync_copy(src_hbm.at[wid, pl.ds(jj * _SC_SK, _SC_SK)], src_v)
            pltpu.sync_copy(dst_hbm.at[wid, pl.ds(jj * _SC_SK, _SC_SK)], dst_v)
            pltpu.sync_copy(ew_hbm.at[wid, pl.ds(jj * _SC_SK, _SC_SK)], ew_v)

            g_cur = pltpu.async_copy(y_hbm.at[src_v.at[0]], rows_v.at[0], gsem)
            sc_prev = None
            for j in range(_SC_SK):
                b = j % 2
                if sc_prev is not None:
                    sc_prev.wait()
                    sc_prev = None
                if j + 1 < _SC_SK:
                    g_next = pltpu.async_copy(y_hbm.at[src_v.at[j + 1]],
                                              rows_v.at[1 - b], gsem)
                g_cur.wait()
                scale(b, j)
                sc_now = pltpu.async_copy(rows_v.at[b], acc.at[dst_v.at[j]],
                                          ssem, add=True)
                if j + 1 < _SC_SK:
                    g_cur = g_next
                    sc_prev = sc_now
                else:
                    sc_now.wait()
            return carry

        lax.fori_loop(0, nsk, superchunk, 0)
        plsc.subcore_barrier()

        for i in range(nz):
            r0 = sid * rps + i * zr
            pltpu.sync_copy(acc.at[pl.ds(r0, zr)],
                            out_hbm.at[cid, pl.ds(r0, zr)])

    return spmm


def _spmm(y, src_r, dst_r, ew_r):
    N = y.shape[0]
    parts = _sc_spmm_make(N, src_r.size)(y, src_r, dst_r, ew_r)
    return parts[0, :N], parts[1, :N]


def _update_kernel(x_ref, a0_ref, a1_ref, wuxh, wuxl, wuah, wual, bu,
                   wph, wpl, bp, x2_out, y2_out):
    agg = a0_ref[...] + a1_ref[...]
    z = (_dot3(x_ref[...], wuxh[...], wuxl[...])
         + _dot3(agg, wuah[...], wual[...]) + bu[...])
    u = _gelu(z)
    s = jnp.maximum(jnp.sum(u * u, axis=1, keepdims=True), 1e-12)
    x2 = u * lax.rsqrt(s)
    x2_out[...] = x2
    y2_out[...] = _gelu(_dot3(x2, wph[...], wpl[...]) + bp[...])


def _run_update(x, a0, a1, wux, wua, bu, wp, bp):
    N = x.shape[0]
    B = 1000 if N % 1000 == 0 else N
    nb = N // B

    def wspec(shape):
        return pl.BlockSpec(shape, lambda i: (0, 0))

    bspec = pl.BlockSpec((B, HID), lambda i: (i, 0))
    return pl.pallas_call(
        _update_kernel,
        grid=(nb,),
        in_specs=[bspec, bspec, bspec,
                  wspec((HID, HID)), wspec((HID, HID)),
                  wspec((HID, HID)), wspec((HID, HID)), wspec((1, HID)),
                  wspec((HID, HID)), wspec((HID, HID)), wspec((1, HID))],
        out_specs=[bspec] * 2,
        out_shape=[jax.ShapeDtypeStruct((N, HID), jnp.float32)] * 2,
    )(x, a0, a1, *wux, *wua, bu, *wp, bp)


def _final_kernel(x_ref, a0_ref, a1_ref, wuxh, wuxl, wuah, wual, bu,
                  wpoh, wpol, bpo, wlh, wll, bl, out_ref):
    agg = a0_ref[...] + a1_ref[...]
    z = (_dot3(x_ref[...], wuxh[...], wuxl[...])
         + _dot3(agg, wuah[...], wual[...]) + bu[...])
    u = _gelu(z)
    s = jnp.maximum(jnp.sum(u * u, axis=1, keepdims=True), 1e-12)
    x3 = u * lax.rsqrt(s)
    emb = _gelu(_dot3(x3, wpoh[...], wpol[...]) + bpo[...])
    out_ref[...] = _dot3(emb, wlh[...], wll[...]) + bl[...]


def _run_final(x, a0, a1, wux, wua, bu, wpo, bpo, wl, bl, ncls):
    N = x.shape[0]
    B = 1000 if N % 1000 == 0 else N
    nb = N // B

    def wspec(shape):
        return pl.BlockSpec(shape, lambda i: (0, 0))

    bspec = pl.BlockSpec((B, HID), lambda i: (i, 0))
    return pl.pallas_call(
        _final_kernel,
        grid=(nb,),
        in_specs=[bspec, bspec, bspec,
                  wspec((HID, HID)), wspec((HID, HID)),
                  wspec((HID, HID)), wspec((HID, HID)), wspec((1, HID)),
                  wspec((HID, HID)), wspec((HID, HID)), wspec((1, HID)),
                  wspec((HID, ncls)), wspec((HID, ncls)), wspec((1, ncls))],
        out_specs=pl.BlockSpec((B, ncls), lambda i: (i, 0)),
        out_shape=jax.ShapeDtypeStruct((N, ncls), jnp.float32),
    )(x, a0, a1, *wux, *wua, bu, *wpo, bpo, *wl, bl)


def kernel(node_features, edges, edge_weights, params):
    edges = edges.astype(jnp.int32)
    N = node_features.shape[0]
    ewn = edge_weights / jnp.sum(edge_weights)
    E = edges.shape[1]
    echunk = _SC_NW * _SC_K * _SC_SK
    epad = (-E) % echunk
    ewn = jnp.pad(ewn, (0, epad))
    src_p = jnp.pad(edges[1], (0, epad))
    dst_p = jnp.pad(edges[0], (0, epad))
    ew_r = ewn.reshape(_SC_NW, -1, _SC_K)
    src_r = src_p.reshape(_SC_NW, -1, _SC_K)
    dst_r = dst_p.reshape(_SC_NW, -1, _SC_K)

    x2d = node_features.reshape(N, P_LEN * PCH).astype(jnp.bfloat16)

    p = params
    wz1 = _split(jnp.concatenate([p["lstm1"]["W"], p["lstm1"]["U"]], axis=0))
    b1 = p["lstm1"]["b"][None, :]
    wz2 = _split(jnp.concatenate([p["lstm2"]["W"], p["lstm2"]["U"]], axis=0))
    b2 = p["lstm2"]["b"][None, :]
    wp1, bp1 = _fold_ffn(p["conv1"]["prepare"])
    wu1, bu1 = _fold_ffn(p["conv1"]["update"])
    wp2, bp2 = _fold_ffn(p["conv2"]["prepare"])
    wu2, bu2 = _fold_ffn(p["conv2"]["update"])
    wpo, bpo = _fold_ffn(p["post"])
    wl = _split(p["logits_W"])
    bl = p["logits_b"][None, :]
    ncls = p["logits_W"].shape[1]

    x1, y1 = _run_lstm(x2d, wz1, b1, wz2, b2, _split(wp1), bp1)
    a0, a1 = _spmm(y1, src_r, dst_r, ew_r)
    x2, y2 = _run_update(x1, a0, a1, _split(wu1[:HID]), _split(wu1[HID:]), bu1,
                         _split(wp2), bp2)
    a0, a1 = _spmm(y2, src_r, dst_r, ew_r)
    return _run_final(x2, a0, a1, _split(wu2[:HID]), _split(wu2[HID:]), bu2,
                      _split(wpo), bpo, wl, bl, ncls)


# speedup vs baseline: 3.0945x; 1.0439x over previous
import functools

import jax
import jax.numpy as jnp
from jax import lax
from jax.experimental import pallas as pl
from jax.experimental.pallas import tpu as pltpu
from jax.experimental.pallas import tpu_sc as plsc

P_LEN = 50
PCH = 26
PCH_PAD = 32
U1 = 64
U2 = 128
HID = 128


def _split(x):
    hi = x.astype(jnp.bfloat16)
    lo = (x - hi.astype(jnp.float32)).astype(jnp.bfloat16)
    return hi, lo


def _dot3(a, bh, bl):
    ah, al = _split(a)
    return (jnp.dot(ah, bh, preferred_element_type=jnp.float32)
            + jnp.dot(ah, bl, preferred_element_type=jnp.float32)
            + jnp.dot(al, bh, preferred_element_type=jnp.float32))


def _gelu(x):
    return 0.5 * x * (1.0 + lax.erf(x * (2.0 ** -0.5)))


def _sig(x):
    return jax.nn.sigmoid(x)


def _fold_ffn(p):
    s = p["bn_g"] / jnp.sqrt(p["bn_v"] + 1e-3)
    t = p["bn_b"] - p["bn_m"] * s
    W = p["W"] * s[:, None]
    b = p["b"] + t @ p["W"]
    return W, b[None, :]


def _dotp(a, b):
    return jnp.dot(a, b, preferred_element_type=jnp.float32)


def _lstm_kernel(x_ref, wz1h, wz1l, b1, wz2h, wz2l, b2,
                 wph, wpl, bp, x1_out, y1_out):
    B = x_ref.shape[0]
    h1 = jnp.zeros((B, U1), jnp.float32)
    c1 = jnp.zeros((B, U1), jnp.float32)
    h2 = jnp.zeros((B, U2), jnp.float32)
    c2 = jnp.zeros((B, U2), jnp.float32)
    b1v, b2v = b1[...], b2[...]
    wz1hv, wz1lv = wz1h[...], wz1l[...]
    wz2hv, wz2lv = wz2h[...], wz2l[...]
    xz = jnp.zeros((B, PCH), jnp.bfloat16)
    for t in range(P_LEN):
        xt = x_ref[:, t * PCH:(t + 1) * PCH]
        h1h, h1l = _split(h1)
        ah1 = jnp.concatenate([xt, h1h], axis=1)
        al1 = jnp.concatenate([xz, h1l], axis=1)
        z1 = _dotp(ah1, wz1hv) + _dotp(ah1, wz1lv) + _dotp(al1, wz1hv) + b1v
        c1 = _sig(z1[:, U1:2 * U1]) * c1 + _sig(z1[:, :U1]) * jnp.tanh(z1[:, 2 * U1:3 * U1])
        h1 = _sig(z1[:, 3 * U1:]) * jnp.tanh(c1)
        h1h, h1l = _split(h1)
        h2h, h2l = _split(h2)
        ah2 = jnp.concatenate([h1h, h2h], axis=1)
        al2 = jnp.concatenate([h1l, h2l], axis=1)
        z2 = _dotp(ah2, wz2hv) + _dotp(ah2, wz2lv) + _dotp(al2, wz2hv) + b2v
        c2 = _sig(z2[:, U2:2 * U2]) * c2 + _sig(z2[:, :U2]) * jnp.tanh(z2[:, 2 * U2:3 * U2])
        h2 = _sig(z2[:, 3 * U2:]) * jnp.tanh(c2)
    x1_out[...] = h2
    y1_out[...] = _gelu(_dot3(h2, wph[...], wpl[...]) + bp[...])


def _run_lstm(x2d, wz1, b1, wz2, b2, wp, bp):
    N = x2d.shape[0]
    B = 2000 if N % 2000 == 0 else (1000 if N % 1000 == 0 else N)
    nb = N // B
    D = P_LEN * PCH

    def wspec(shape):
        return pl.BlockSpec(shape, lambda i: (0, 0))

    return pl.pallas_call(
        _lstm_kernel,
        grid=(nb,),
        in_specs=[
            pl.BlockSpec((B, D), lambda i: (i, 0)),
            wspec((PCH + U1, 4 * U1)), wspec((PCH + U1, 4 * U1)), wspec((1, 4 * U1)),
            wspec((U1 + U2, 4 * U2)), wspec((U1 + U2, 4 * U2)), wspec((1, 4 * U2)),
            wspec((HID, HID)), wspec((HID, HID)), wspec((1, HID)),
        ],
        out_specs=[pl.BlockSpec((B, HID), lambda i: (i, 0))] * 2,
        out_shape=[jax.ShapeDtypeStruct((N, HID), jnp.float32)] * 2,
    )(x2d, *wz1, b1, *wz2, b2, *wp, bp)


_SC_NC = 2
_SC_NS = 16
_SC_NW = _SC_NC * _SC_NS
_SC_K = 128
_SC_SK = 8


@functools.lru_cache(maxsize=None)
def _sc_spmm_make(N, E):
    epw = E // _SC_NW
    nch = epw // _SC_K
    nsk = nch // _SC_SK
    zr = 64
    npad = ((N + zr * _SC_NS - 1) // (zr * _SC_NS)) * (zr * _SC_NS)
    rps = npad // _SC_NS
    nz = rps // zr
    mesh = plsc.VectorSubcoreMesh(core_axis_name="c", subcore_axis_name="s")

    @functools.partial(
        pl.kernel,
        out_type=jax.ShapeDtypeStruct((_SC_NC, npad, HID), jnp.float32),
        mesh=mesh,
        scratch_types=[
            pltpu.VMEM((_SC_SK, _SC_K), jnp.int32),
            pltpu.VMEM((_SC_SK, _SC_K), jnp.int32),
            pltpu.VMEM((_SC_SK, _SC_K), jnp.float32),
            pltpu.VMEM((2, _SC_K, HID), jnp.float32),
            pltpu.VMEM((zr, HID), jnp.float32),
            pltpu.VMEM_SHARED((npad, HID), jnp.float32),
            pltpu.SemaphoreType.DMA,
            pltpu.SemaphoreType.DMA,
        ],
    )
    def spmm(y_hbm, src_hbm, dst_hbm, ew_hbm, out_hbm,
             src_v, dst_v, ew_v, rows_v, zbuf, acc, gsem, ssem):
        cid = lax.axis_index("c")
        sid = lax.axis_index("s")
        wid = sid * _SC_NC + cid

        zv = jnp.zeros((16,), jnp.float32)

        def zrow(r, carry):
            for k8 in range(HID // 16):
                zbuf[r, pl.ds(16 * k8, 16)] = zv
            return carry

        lax.fori_loop(0, zr, zrow, 0)
        for i in range(nz):
            pltpu.sync_copy(zbuf, acc.at[pl.ds(sid * rps + i * zr, zr)])
        plsc.subcore_barrier()

        def scale(b, j):
            def edge16(g, c2):
                wg = ew_v[j, pl.ds(16 * g, 16)]
                for m in range(16):
                    w = lax.gather(
                        wg, jnp.full((16, 1), m, jnp.int32),
                        lax.GatherDimensionNumbers(
                            offset_dims=(), collapsed_slice_dims=(0,),
                            start_index_map=(0,)),
                        (1,),
                        mode=lax.GatherScatterMode.PROMISE_IN_BOUNDS)
                    e = g * 16 + m
                    for k8 in range(HID // 16):
                        s = pl.ds(16 * k8, 16)
                        rows_v[b, e, s] = rows_v[b, e, s] * w
                return c2

            lax.fori_loop(0, _SC_K // 16, edge16, 0)

        def superchunk(jj, carry):
            pltpu.sync_copy(src_hbm.at[wid, pl.ds(jj * _SC_SK, _SC_SK)], src_v)
            pltpu.sync_copy(dst_hbm.at[wid, pl.ds(jj * _SC_SK, _SC_SK)], dst_v)
            pltpu.sync_copy(ew_hbm.at[wid, pl.ds(jj * _SC_SK, _SC_SK)], ew_v)

            g_cur = pltpu.async_copy(y_hbm.at[src_v.at[0]], rows_v.at[0], gsem)
            sc_prev = None
            for j in range(_SC_SK):
                b = j % 2
                if sc_prev is not None:
                    sc_prev.wait()
                    sc_prev = None
                if j + 1 < _SC_SK:
                    g_next = pltpu.async_copy(y_hbm.at[src_v.at[j + 1]],
                                              rows_v.at[1 - b], gsem)
                g_cur.wait()
                scale(b, j)
                sc_now = pltpu.async_copy(rows_v.at[b], acc.at[dst_v.at[j]],
                                          ssem, add=True)
                if j + 1 < _SC_SK:
                    g_cur = g_next
                    sc_prev = sc_now
                else:
                    sc_now.wait()
            return carry

        lax.fori_loop(0, nsk, superchunk, 0)
        plsc.subcore_barrier()

        for i in range(nz):
            r0 = sid * rps + i * zr
            pltpu.sync_copy(acc.at[pl.ds(r0, zr)],
                            out_hbm.at[cid, pl.ds(r0, zr)])

    return spmm


def _spmm(y, src_r, dst_r, ew_r):
    N = y.shape[0]
    parts = _sc_spmm_make(N, src_r.size)(y, src_r, dst_r, ew_r)
    return parts[0, :N], parts[1, :N]


def _update_kernel(x_ref, a0_ref, a1_ref, wuxh, wuxl, wuah, wual, bu,
                   wph, wpl, bp, x2_out, y2_out):
    agg = a0_ref[...] + a1_ref[...]
    z = (_dot3(x_ref[...], wuxh[...], wuxl[...])
         + _dot3(agg, wuah[...], wual[...]) + bu[...])
    u = _gelu(z)
    s = jnp.maximum(jnp.sum(u * u, axis=1, keepdims=True), 1e-12)
    x2 = u * lax.rsqrt(s)
    x2_out[...] = x2
    y2_out[...] = _gelu(_dot3(x2, wph[...], wpl[...]) + bp[...])


def _run_update(x, a0, a1, wux, wua, bu, wp, bp):
    N = x.shape[0]
    B = 1000 if N % 1000 == 0 else N
    nb = N // B

    def wspec(shape):
        return pl.BlockSpec(shape, lambda i: (0, 0))

    bspec = pl.BlockSpec((B, HID), lambda i: (i, 0))
    return pl.pallas_call(
        _update_kernel,
        grid=(nb,),
        in_specs=[bspec, bspec, bspec,
                  wspec((HID, HID)), wspec((HID, HID)),
                  wspec((HID, HID)), wspec((HID, HID)), wspec((1, HID)),
                  wspec((HID, HID)), wspec((HID, HID)), wspec((1, HID))],
        out_specs=[bspec] * 2,
        out_shape=[jax.ShapeDtypeStruct((N, HID), jnp.float32)] * 2,
    )(x, a0, a1, *wux, *wua, bu, *wp, bp)


def _final_kernel(x_ref, a0_ref, a1_ref, wuxh, wuxl, wuah, wual, bu,
                  wpoh, wpol, bpo, wlh, wll, bl, out_ref):
    agg = a0_ref[...] + a1_ref[...]
    z = (_dot3(x_ref[...], wuxh[...], wuxl[...])
         + _dot3(agg, wuah[...], wual[...]) + bu[...])
    u = _gelu(z)
    s = jnp.maximum(jnp.sum(u * u, axis=1, keepdims=True), 1e-12)
    x3 = u * lax.rsqrt(s)
    emb = _gelu(_dot3(x3, wpoh[...], wpol[...]) + bpo[...])
    out_ref[...] = _dot3(emb, wlh[...], wll[...]) + bl[...]


def _run_final(x, a0, a1, wux, wua, bu, wpo, bpo, wl, bl, ncls):
    N = x.shape[0]
    B = 1000 if N % 1000 == 0 else N
    nb = N // B

    def wspec(shape):
        return pl.BlockSpec(shape, lambda i: (0, 0))

    bspec = pl.BlockSpec((B, HID), lambda i: (i, 0))
    return pl.pallas_call(
        _final_kernel,
        grid=(nb,),
        in_specs=[bspec, bspec, bspec,
                  wspec((HID, HID)), wspec((HID, HID)),
                  wspec((HID, HID)), wspec((HID, HID)), wspec((1, HID)),
                  wspec((HID, HID)), wspec((HID, HID)), wspec((1, HID)),
                  wspec((HID, ncls)), wspec((HID, ncls)), wspec((1, ncls))],
        out_specs=pl.BlockSpec((B, ncls), lambda i: (i, 0)),
        out_shape=jax.ShapeDtypeStruct((N, ncls), jnp.float32),
    )(x, a0, a1, *wux, *wua, bu, *wpo, bpo, *wl, bl)


def kernel(node_features, edges, edge_weights, params):
    edges = edges.astype(jnp.int32)
    N = node_features.shape[0]
    ewn = edge_weights / jnp.sum(edge_weights)
    E = edges.shape[1]
    echunk = _SC_NW * _SC_K * _SC_SK
    epad = (-E) % echunk
    ewn = jnp.pad(ewn, (0, epad))
    src_p = jnp.pad(edges[1], (0, epad))
    dst_p = jnp.pad(edges[0], (0, epad))
    ew_r = ewn.reshape(_SC_NW, -1, _SC_K)
    src_r = src_p.reshape(_SC_NW, -1, _SC_K)
    dst_r = dst_p.reshape(_SC_NW, -1, _SC_K)

    x2d = node_features.reshape(N, P_LEN * PCH).astype(jnp.bfloat16)

    p = params
    wz1 = _split(jnp.concatenate([p["lstm1"]["W"], p["lstm1"]["U"]], axis=0))
    b1 = p["lstm1"]["b"][None, :]
    wz2 = _split(jnp.concatenate([p["lstm2"]["W"], p["lstm2"]["U"]], axis=0))
    b2 = p["lstm2"]["b"][None, :]
    wp1, bp1 = _fold_ffn(p["conv1"]["prepare"])
    wu1, bu1 = _fold_ffn(p["conv1"]["update"])
    wp2, bp2 = _fold_ffn(p["conv2"]["prepare"])
    wu2, bu2 = _fold_ffn(p["conv2"]["update"])
    wpo, bpo = _fold_ffn(p["post"])
    wl = _split(p["logits_W"])
    bl = p["logits_b"][None, :]
    ncls = p["logits_W"].shape[1]

    x1, y1 = _run_lstm(x2d, wz1, b1, wz2, b2, _split(wp1), bp1)
    a0, a1 = _spmm(y1, src_r, dst_r, ew_r)
    x2, y2 = _run_update(x1, a0, a1, _split(wu1[:HID]), _split(wu1[HID:]), bu1,
                         _split(wp2), bp2)
    a0, a1 = _spmm(y2, src_r, dst_r, ew_r)
    return _run_final(x2, a0, a1, _split(wu2[:HID]), _split(wu2[HID:]), bu2,
                      _split(wpo), bpo, wl, bl, ncls)
